# Initial kernel scaffold; baseline (speedup 1.0000x reference)
import jax
import jax.numpy as jnp
from jax.experimental import pallas as pl


def kernel(node_feature, edge_index, table, Wh, bh, Wfp, bfp, Wcl, bcl):
    raise NotImplementedError("write your pallas kernel here")


# speedup vs baseline: 4.7057x; 4.7057x over previous
import functools

import jax
import jax.numpy as jnp
from jax import lax
from jax.experimental import pallas as pl
from jax.experimental.pallas import tpu as pltpu
from jax.experimental.pallas import tpu_sc as plsc

N = 10000
E = 160000
NUM_FEAT = 128
F = 256
R = 3
L = 512
C = 16

NC = 2
NS = 16
HALF = F // NC

NP = 10240
EPT = E // NS
ECH = 80
NECH = EPT // ECH
NPT = NP // NS
RCH = 80
NRCH = NPT // RCH

_mesh = plsc.VectorSubcoreMesh(core_axis_name="c", subcore_axis_name="s")


@functools.partial(
    pl.kernel,
    out_type=jax.ShapeDtypeStruct((NC, NP, HALF), jnp.float32),
    mesh=_mesh,
    scratch_types=[
        pltpu.VMEM((NRCH, RCH), jnp.int32),
        pltpu.VMEM((RCH, HALF), jnp.float32),
        pltpu.SemaphoreType.DMA,
    ],
)
def _sc_embed(feat_hbm, table_hbm, emb_hbm, idx_v, rows_v, sem):
    c = lax.axis_index("c")
    s = lax.axis_index("s")
    pltpu.sync_copy(feat_hbm.at[s], idx_v)
    for j in range(NRCH):
        pltpu.async_copy(table_hbm.at[c].at[idx_v.at[j]], rows_v, sem).wait()
        pltpu.sync_copy(rows_v, emb_hbm.at[c].at[pl.ds(s * NPT + j * RCH, RCH)])


@functools.partial(
    pl.kernel,
    out_type=jax.ShapeDtypeStruct((NC, NP, HALF), jnp.float32),
    mesh=_mesh,
    scratch_types=[
        pltpu.VMEM((NECH, ECH), jnp.int32),
        pltpu.VMEM((NECH, ECH), jnp.int32),
        pltpu.VMEM((ECH, HALF), jnp.float32),
        pltpu.VMEM_SHARED((NP, HALF), jnp.float32),
        pltpu.SemaphoreType.DMA,
    ],
)
def _sc_segsum(emb_hbm, src_hbm, dst_hbm, agg_hbm,
               sidx, didx, rows, acc, sem):
    c = lax.axis_index("c")
    s = lax.axis_index("s")
    base = s * NPT
    pltpu.sync_copy(src_hbm.at[s], sidx)
    pltpu.sync_copy(dst_hbm.at[s], didx)
    for j in range(NRCH):
        pltpu.sync_copy(emb_hbm.at[c].at[pl.ds(base + j * RCH, RCH)], rows)
        pltpu.sync_copy(rows, acc.at[pl.ds(base + j * RCH, RCH)])
    plsc.subcore_barrier()

    def body(j, carry):
        pltpu.async_copy(emb_hbm.at[c].at[sidx.at[j]], rows, sem).wait()
        pltpu.sync_copy(rows, acc.at[didx.at[j]], add=True)
        return carry

    lax.fori_loop(0, NECH, body, 0)
    plsc.subcore_barrier()
    for j in range(NRCH):
        pltpu.sync_copy(acc.at[pl.ds(base + j * RCH, RCH)], rows)
        pltpu.sync_copy(rows, agg_hbm.at[c].at[pl.ds(base + j * RCH, RCH)])


BLK = 1000
_DN = (((1,), (1,)), ((), ()))


def _tc_round_body(agg_ref, wh_ref, bh_ref, wfp_ref, bfp_ref, h_ref, f_ref):
    h = lax.dot_general(agg_ref[0], wh_ref[0], _DN,
                        preferred_element_type=jnp.float32)
    h = h + lax.dot_general(agg_ref[1], wh_ref[1], _DN,
                            preferred_element_type=jnp.float32)
    h = jnp.maximum(h + bh_ref[...], 0.0)
    h_ref[0] = h[:, :HALF]
    h_ref[1] = h[:, HALF:]
    lg = lax.dot_general(h[:, :HALF], wfp_ref[0], _DN,
                         preferred_element_type=jnp.float32)
    lg = lg + lax.dot_general(h[:, HALF:], wfp_ref[1], _DN,
                              preferred_element_type=jnp.float32)
    lg = lg + bfp_ref[...]
    m = jnp.max(lg, axis=-1, keepdims=True)
    e = jnp.exp(lg - m)
    p = e / jnp.sum(e, axis=-1, keepdims=True)

    @pl.when(pl.program_id(0) == 0)
    def _init():
        f_ref[...] = jnp.zeros_like(f_ref)

    f_ref[...] += jnp.sum(p, axis=0, keepdims=True)


def _tc_round(agg3, wh3, bh2, wfp3, bfp2):
    return pl.pallas_call(
        _tc_round_body,
        grid=(N // BLK,),
        in_specs=[
            pl.BlockSpec((NC, BLK, HALF), lambda i: (0, i, 0)),
            pl.BlockSpec((NC, F, HALF), lambda i: (0, 0, 0)),
            pl.BlockSpec((1, F), lambda i: (0, 0)),
            pl.BlockSpec((NC, L, HALF), lambda i: (0, 0, 0)),
            pl.BlockSpec((1, L), lambda i: (0, 0)),
        ],
        out_specs=[
            pl.BlockSpec((NC, BLK, HALF), lambda i: (0, i, 0)),
            pl.BlockSpec((1, L), lambda i: (0, 0)),
        ],
        out_shape=[
            jax.ShapeDtypeStruct((NC, NP, HALF), jnp.float32),
            jax.ShapeDtypeStruct((1, L), jnp.float32),
        ],
    )(agg3, wh3, bh2, wfp3, bfp2)


def _tc_final_body(f0_ref, f1_ref, f2_ref, wcl_ref, bcl_ref, out_ref):
    f = f0_ref[...] + f1_ref[...] + f2_ref[...]
    lg = lax.dot_general(f, wcl_ref[...], _DN,
                         preferred_element_type=jnp.float32) + bcl_ref[...]
    m = jnp.max(lg)
    e = jnp.exp(lg - m)
    out_ref[...] = e / jnp.sum(e)


def _tc_final(f0, f1, f2, wcl, bcl2):
    return pl.pallas_call(
        _tc_final_body,
        out_shape=jax.ShapeDtypeStruct((1, C), jnp.float32),
    )(f0, f1, f2, wcl, bcl2)


def kernel(node_feature, edge_index, table, Wh, bh, Wfp, bfp, Wcl, bcl):
    feat = jnp.concatenate(
        [node_feature.astype(jnp.int32),
         jnp.zeros((NP - N,), jnp.int32)]).reshape(NS, NRCH, RCH)
    src = edge_index[0].astype(jnp.int32).reshape(NS, NECH, ECH)
    dst = edge_index[1].astype(jnp.int32).reshape(NS, NECH, ECH)
    table3 = table.reshape(NUM_FEAT, NC, HALF).transpose(1, 0, 2)
    wh3 = Wh.reshape(R, F, NC, HALF).transpose(0, 2, 1, 3)
    wfp3 = Wfp.reshape(R, L, NC, HALF).transpose(0, 2, 1, 3)

    emb = _sc_embed(feat, table3)
    fparts = []
    for r in range(R):
        agg = _sc_segsum(emb, src, dst)
        emb, fp = _tc_round(agg, wh3[r], bh[r].reshape(1, F),
                            wfp3[r], bfp[r].reshape(1, L))
        fparts.append(fp)
    out = _tc_final(fparts[0], fparts[1], fparts[2], Wcl, bcl.reshape(1, C))
    return out.reshape(C)
